# baseline (speedup 1.0000x reference)
import functools
import math

import jax
import jax.numpy as jnp
from jax import lax
from jax.experimental import pallas as pl
from jax.experimental.pallas import tpu as pltpu
from jax.experimental.pallas import tpu_sc as plsc

N_NODES = 10000
N_EDGES = 160000
NUM_GRAPHS = 64
D = 256
H = 128

NPAD = 10240
EPT = 10240
EPAD = 16 * EPT
KE = 64
NB = EPT // KE
IC = 40
NCH = NB // IC
NZ = NPAD // 16
ZC = NZ // KE


def _gelu(x):
    return 0.5 * x * (1.0 + lax.erf(x * (1.0 / math.sqrt(2.0))))


def _ln(x, g, b):
    mu = jnp.mean(x, axis=-1, keepdims=True)
    var = jnp.var(x, axis=-1, keepdims=True)
    return (x - mu) * jax.lax.rsqrt(var + 1e-5) * g + b


def _node_encoder_body(xb_ref, x62_ref, xesm_ref,
                       base_g, base_b, b1w, b1b, b2w, b2b,
                       s62_g, s62_b, s1w, s1b, s2w, s2b,
                       esm_g, esm_b, e1w, e1b, e2w, e2b,
                       st_g, st_b, stw, stb,
                       g1w, g1b, g2w, g2b,
                       fu_g, fu_b, fuw, fub,
                       out_ref):
    f32 = jnp.float32
    xb = xb_ref[...]
    x62 = x62_ref[...]
    xesm = xesm_ref[...]

    h = _ln(xb, base_g[...], base_b[...])
    h = _gelu(jnp.dot(h, b1w[...], preferred_element_type=f32) + b1b[...])
    h_base = _gelu(jnp.dot(h, b2w[...], preferred_element_type=f32) + b2b[...])

    h = _ln(x62, s62_g[...], s62_b[...])
    h = _gelu(jnp.dot(h, s1w[...], preferred_element_type=f32) + s1b[...])
    h_b62 = _gelu(jnp.dot(h, s2w[...], preferred_element_type=f32) + s2b[...])

    hs = _ln(jnp.concatenate([h_base, h_b62], axis=1), st_g[...], st_b[...])
    h_struct = _gelu(jnp.dot(hs, stw[...], preferred_element_type=f32) + stb[...])

    h = _ln(xesm, esm_g[...], esm_b[...])
    h = _gelu(jnp.dot(h, e1w[...], preferred_element_type=f32) + e1b[...])
    h_esm = _gelu(jnp.dot(h, e2w[...], preferred_element_type=f32) + e2b[...])

    cat = jnp.concatenate([h_struct, h_esm], axis=1)
    g = _gelu(jnp.dot(cat, g1w[...], preferred_element_type=f32) + g1b[...])
    gate = jax.nn.sigmoid(jnp.dot(g, g2w[...], preferred_element_type=f32) + g2b[...])
    h_esm = h_esm * gate

    cat2 = jnp.concatenate([h_struct, h_esm], axis=1)
    fu = _ln(cat2, fu_g[...], fu_b[...])
    y = _gelu(jnp.dot(fu, fuw[...], preferred_element_type=f32) + fub[...])
    out_ref[0] = y[:, :H]
    out_ref[1] = y[:, H:]


def _node_encoder(x_base, x_b62, x_esm, p):
    BN = 1000
    n_blocks = N_NODES // BN

    def row(d):
        return d.reshape(1, -1)

    weights = [
        row(p["base_ln"]["g"]), row(p["base_ln"]["b"]),
        p["base_l1"]["W"].T, row(p["base_l1"]["b"]),
        p["base_l2"]["W"].T, row(p["base_l2"]["b"]),
        row(p["b62_ln"]["g"]), row(p["b62_ln"]["b"]),
        p["b62_l1"]["W"].T, row(p["b62_l1"]["b"]),
        p["b62_l2"]["W"].T, row(p["b62_l2"]["b"]),
        row(p["esm_ln"]["g"]), row(p["esm_ln"]["b"]),
        p["esm_l1"]["W"].T, row(p["esm_l1"]["b"]),
        p["esm_l2"]["W"].T, row(p["esm_l2"]["b"]),
        row(p["struct_ln"]["g"]), row(p["struct_ln"]["b"]),
        p["struct_l"]["W"].T, row(p["struct_l"]["b"]),
        p["gate_l1"]["W"].T, row(p["gate_l1"]["b"]),
        p["gate_l2"]["W"].T, row(p["gate_l2"]["b"]),
        row(p["fuse_ln"]["g"]), row(p["fuse_ln"]["b"]),
        p["fuse_l"]["W"].T, row(p["fuse_l"]["b"]),
    ]

    in_specs = [
        pl.BlockSpec((BN, 12), lambda i: (i, 0)),
        pl.BlockSpec((BN, 20), lambda i: (i, 0)),
        pl.BlockSpec((BN, 1280), lambda i: (i, 0)),
    ] + [pl.BlockSpec(w.shape, lambda i: (0, 0)) for w in weights]

    return pl.pallas_call(
        _node_encoder_body,
        grid=(n_blocks,),
        in_specs=in_specs,
        out_specs=pl.BlockSpec((2, BN, H), lambda i: (0, i, 0)),
        out_shape=jax.ShapeDtypeStruct((2, N_NODES, H), jnp.float32),
    )(x_base, x_b62, x_esm, *weights)


def _make_edge_lins_body(n_lins):
    def body(dist_ref, seqbin_ref, isseq_ref, invd_ref,
             semb_ref, w1_ref, b1_ref, w2_ref, b2_ref, *rest):
        lin_ws = rest[:-1]
        out_ref = rest[-1]
        f32 = jnp.float32
        bf16 = jnp.bfloat16
        B = dist_ref.shape[0]
        dist = dist_ref[...]
        seqbin = seqbin_ref[...]
        isseq = isseq_ref[...]
        invd = invd_ref[...]

        centers = lax.broadcasted_iota(jnp.int32, (B, 32), 1).astype(f32) * (20.0 / 31.0)
        widths = 20.0 / 32.0
        gamma = 1.0 / (widths * widths + 1e-08)
        rbf = jnp.exp(-gamma * (dist - centers) ** 2)

        onehot = (seqbin == lax.broadcasted_iota(jnp.int32, (B, 9), 1)).astype(f32)
        seq_feat = jnp.dot(onehot, semb_ref[...], preferred_element_type=f32)

        raw = jnp.concatenate([rbf, seq_feat, isseq, invd], axis=1)
        hmid = _gelu(jnp.dot(raw, w1_ref[...], preferred_element_type=f32) + b1_ref[...])
        attr = jnp.dot(hmid, w2_ref[...], preferred_element_type=f32) + b2_ref[...]
        attr16 = attr.astype(bf16)

        for l in range(n_lins):
            lw, lb = lin_ws[2 * l], lin_ws[2 * l + 1]
            el = jnp.dot(attr16, lw[...], preferred_element_type=f32) + lb[...]
            out_ref[l, 0] = el[:, :H]
            out_ref[l, 1] = el[:, H:]
    return body


def _edge_lins(edge_dist, edge_seqbin, edge_is_seq, edge_inv_dist, p, layers):
    BE = 2048
    n_blocks = EPAD // BE
    col = lambda a: a.reshape(-1, 1)
    weights = [
        p["seq_emb"],
        p["edge_l1"]["W"].T, p["edge_l1"]["b"].reshape(1, -1),
        p["edge_l2"]["W"].T, p["edge_l2"]["b"].reshape(1, -1),
    ]
    for l in layers:
        c = p["convs"][l]
        weights += [c["lin"]["W"].T.astype(jnp.bfloat16), c["lin"]["b"].reshape(1, -1)]
    in_specs = [pl.BlockSpec((BE, 1), lambda i: (i, 0))] * 4 + [
        pl.BlockSpec(w.shape, lambda i: (0, 0)) for w in weights
    ]
    nl = len(layers)
    return pl.pallas_call(
        _make_edge_lins_body(nl),
        grid=(n_blocks,),
        in_specs=in_specs,
        out_specs=pl.BlockSpec((nl, 2, BE, H), lambda i: (0, 0, i, 0)),
        out_shape=jax.ShapeDtypeStruct((nl, 2, EPAD, H), jnp.float32),
    )(col(edge_dist), col(edge_seqbin.astype(jnp.int32)),
      col(edge_is_seq), col(edge_inv_dist), *weights)


def _make_mp_kernel():
    mesh = plsc.VectorSubcoreMesh(core_axis_name="c", subcore_axis_name="s")

    @functools.partial(
        pl.kernel,
        out_type=jax.ShapeDtypeStruct((2 * NPAD, H), jnp.float32),
        mesh=mesh,
        scratch_types=[
            pltpu.VMEM((IC * KE,), jnp.int32),
            pltpu.VMEM((IC, KE), jnp.int32),
            pltpu.VMEM((KE, H), jnp.float32),
            pltpu.VMEM((KE, H), jnp.float32),
            pltpu.VMEM((KE, H), jnp.float32),
            pltpu.VMEM((KE, H), jnp.float32),
            pltpu.VMEM_SHARED((NPAD, H), jnp.float32),
            pltpu.SemaphoreType.DMA,
            pltpu.SemaphoreType.DMA,
            pltpu.SemaphoreType.DMA,
            pltpu.SemaphoreType.DMA,
            pltpu.SemaphoreType.DMA,
            pltpu.SemaphoreType.DMA,
        ],
    )
    def mp(xsc, el, src2, dst3, out,
           src_i, dst_i, g0, g1, e0, e1, acc, sg0, sg1, se0, se1, ss0, ss1):
        c = lax.axis_index("c")
        s = lax.axis_index("s")
        off = c * N_NODES
        MASKHI = jnp.int32(-65536)

        zero = jnp.zeros((16,), jnp.float32)

        def z_body(r, carry):
            for t in range(H // 16):
                g0[r, pl.ds(t * 16, 16)] = zero
            return carry
        lax.fori_loop(0, KE, z_body, 0)
        for i in range(ZC):
            pltpu.sync_copy(g0, acc.at[pl.ds(s * NZ + i * KE, KE)])
        plsc.subcore_barrier()

        el_base = c * EPAD + s * EPT

        def load_idx(k):
            pltpu.sync_copy(src2.at[s].at[pl.ds(k * IC * KE, IC * KE)], src_i)

            def adj_body(r, carry):
                sl = pl.ds(r * 16, 16)
                src_i[sl] = src_i[sl] + off
                return carry
            lax.fori_loop(0, IC * KE // 16, adj_body, 0)
            pltpu.sync_copy(dst3.at[s].at[pl.ds(k * IC, IC)], dst_i)

        def start(jl, g, e, sg, se, k):
            pltpu.async_copy(xsc.at[src_i.at[pl.ds(jl * KE, KE)]], g, sg)
            pltpu.async_copy(
                el.at[pl.ds(el_base + (k * IC + jl) * KE, KE)], e, se)

        def finish(jl, g, e, sg, se, ss, k):
            pltpu.make_async_copy(xsc.at[src_i.at[pl.ds(jl * KE, KE)]], g, sg).wait()
            pltpu.make_async_copy(
                el.at[pl.ds(el_base + (k * IC + jl) * KE, KE)], e, se).wait()

            def cmp_body(q, carry):
                r = 2 * q
                for rr in (r, r + 1):
                    for t in range(H // 16):
                        sl = (rr, pl.ds(t * 16, 16))
                        g[sl] = jnp.maximum(g[sl] + e[sl], 0.0)
                return carry
            lax.fori_loop(0, KE // 2, cmp_body, 0)
            pltpu.async_copy(g, acc.at[dst_i.at[jl]], ss, add=True)

        for k in range(NCH):
            load_idx(k)
            start(0, g0, e0, sg0, se0, k)
            start(1, g1, e1, sg1, se1, k)

            def pair_body(t, carry):
                jl = 2 * t
                jl1 = 2 * t + 1
                finish(jl, g0, e0, sg0, se0, ss0, k)
                finish(jl1, g1, e1, sg1, se1, ss1, k)
                pltpu.make_async_copy(g0, acc.at[dst_i.at[jl]], ss0).wait()

                @pl.when(jl + 2 < IC)
                def _():
                    start(jl + 2, g0, e0, sg0, se0, k)

                pltpu.make_async_copy(g1, acc.at[dst_i.at[jl1]], ss1).wait()

                @pl.when(jl1 + 2 < IC)
                def _():
                    start(jl1 + 2, g1, e1, sg1, se1, k)
                return carry
            lax.fori_loop(0, IC // 2, pair_body, 0)
        plsc.subcore_barrier()

        for i in range(ZC):
            pltpu.sync_copy(acc.at[pl.ds(s * NZ + i * KE, KE)],
                            out.at[pl.ds(c * NPAD + s * NZ + i * KE, KE)])

    return mp


_MP = _make_mp_kernel()


import numpy as _np
PERM_I = _np.empty(D, dtype=_np.int32)
for _g in range(D // 32):
    for _j in range(16):
        PERM_I[_g * 32 + 2 * _j] = _g * 32 + _j
        PERM_I[_g * 32 + 2 * _j + 1] = _g * 32 + 16 + _j


def _node_mlp_body(x_ref, agg_ref, w1, b1, w2, b2, g_ref, be_ref,
                   out_ref):
    f32 = jnp.float32
    x = jnp.concatenate([x_ref[0], x_ref[1]], axis=1)
    agg = jnp.concatenate([agg_ref[0], agg_ref[1]], axis=1)
    h = _gelu(jnp.dot(x + agg, w1[...], preferred_element_type=f32) + b1[...])
    y = jnp.dot(h, w2[...], preferred_element_type=f32) + b2[...]
    y = _ln(y, g_ref[...], be_ref[...])
    y = _gelu(y)
    y = y + x
    out_ref[0] = y[:, :H]
    out_ref[1] = y[:, H:]


def _node_mlp(x2, aggp, c, nrm):
    BN = 2000
    n_blocks = N_NODES // BN
    weights = [
        c["nn1"]["W"].T, c["nn1"]["b"].reshape(1, -1),
        c["nn2"]["W"].T, c["nn2"]["b"].reshape(1, -1),
        nrm["g"].reshape(1, -1), nrm["b"].reshape(1, -1),
    ]
    in_specs = [
        pl.BlockSpec((2, BN, H), lambda i: (0, i, 0)),
        pl.BlockSpec((2, BN, H), lambda i: (0, i, 0)),
    ] + [pl.BlockSpec(w.shape, lambda i: (0, 0)) for w in weights]
    return pl.pallas_call(
        _node_mlp_body,
        grid=(n_blocks,),
        in_specs=in_specs,
        out_specs=pl.BlockSpec((2, BN, H), lambda i: (0, i, 0)),
        out_shape=jax.ShapeDtypeStruct((2, N_NODES, H), jnp.float32),
    )(x2, aggp, *weights)


def kernel(x_base, x_b62, x_esm, edge_dist, edge_seqbin, edge_is_seq,
           edge_inv_dist, edge_index, batch, params):
    p = params

    def padv(a, value=0.0):
        return jnp.pad(a.reshape(16, N_EDGES // 16), ((0, 0), (0, EPT - N_EDGES // 16)),
                       constant_values=value).reshape(-1)

    x2 = _node_encoder(x_base, x_b62, x_esm, p)
    ed, es, ei, ev = (padv(edge_dist), padv(edge_seqbin.astype(jnp.int32)),
                      padv(edge_is_seq), padv(edge_inv_dist))
    el_all = _edge_lins(ed, es, ei, ev, p, [0, 1, 2, 3, 4])

    src2 = padv(edge_index[0].astype(jnp.int32), 0).reshape(16, EPT)
    dst3 = padv(edge_index[1].astype(jnp.int32), NPAD - 1).reshape(16, NB, KE)

    el_flat = el_all.reshape(5, 2 * EPAD, H)

    for l, (c, nrm) in enumerate(zip(p["convs"], p["norms"])):
        xsc = x2.reshape(2 * N_NODES, H)
        aggp = _MP(xsc, el_flat[l], src2, dst3)
        x2 = _node_mlp(x2, aggp.reshape(2, NPAD, H), c, nrm)

    x = jnp.concatenate([x2[0], x2[1]], axis=1)
    ones = jnp.ones((N_NODES, 1), jnp.float32)
    cnt = jax.ops.segment_sum(ones, batch, num_segments=NUM_GRAPHS)
    h_mean = jax.ops.segment_sum(x, batch, num_segments=NUM_GRAPHS) / jnp.clip(cnt, 1.0, None)
    h_max = jax.ops.segment_max(x, batch, num_segments=NUM_GRAPHS)
    pooled = jnp.concatenate([h_mean, h_max], axis=1)
    return _gelu(pooled @ p["readout"]["W"].T + p["readout"]["b"])


# speedup vs baseline: 1.0511x; 1.0511x over previous
import functools
import math

import jax
import jax.numpy as jnp
from jax import lax
from jax.experimental import pallas as pl
from jax.experimental.pallas import tpu as pltpu
from jax.experimental.pallas import tpu_sc as plsc

N_NODES = 10000
N_EDGES = 160000
NUM_GRAPHS = 64
D = 256
H = 128

NPAD = 10240
EPT = 10240
EPAD = 16 * EPT
KE = 64
NB = EPT // KE
IC = 40
NCH = NB // IC
NZ = NPAD // 16
ZC = NZ // KE


def _gelu(x):
    return 0.5 * x * (1.0 + lax.erf(x * (1.0 / math.sqrt(2.0))))


def _ln(x, g, b):
    mu = jnp.mean(x, axis=-1, keepdims=True)
    var = jnp.var(x, axis=-1, keepdims=True)
    return (x - mu) * jax.lax.rsqrt(var + 1e-5) * g + b


def _node_encoder_body(xb_ref, x62_ref, xesm_ref,
                       base_g, base_b, b1w, b1b, b2w, b2b,
                       s62_g, s62_b, s1w, s1b, s2w, s2b,
                       esm_g, esm_b, e1w, e1b, e2w, e2b,
                       st_g, st_b, stw, stb,
                       g1w, g1b, g2w, g2b,
                       fu_g, fu_b, fuw, fub,
                       out_ref):
    f32 = jnp.float32
    xb = xb_ref[...]
    x62 = x62_ref[...]
    xesm = xesm_ref[...]

    h = _ln(xb, base_g[...], base_b[...])
    h = _gelu(jnp.dot(h, b1w[...], preferred_element_type=f32) + b1b[...])
    h_base = _gelu(jnp.dot(h, b2w[...], preferred_element_type=f32) + b2b[...])

    h = _ln(x62, s62_g[...], s62_b[...])
    h = _gelu(jnp.dot(h, s1w[...], preferred_element_type=f32) + s1b[...])
    h_b62 = _gelu(jnp.dot(h, s2w[...], preferred_element_type=f32) + s2b[...])

    hs = _ln(jnp.concatenate([h_base, h_b62], axis=1), st_g[...], st_b[...])
    h_struct = _gelu(jnp.dot(hs, stw[...], preferred_element_type=f32) + stb[...])

    h = _ln(xesm, esm_g[...], esm_b[...])
    h = _gelu(jnp.dot(h, e1w[...], preferred_element_type=f32) + e1b[...])
    h_esm = _gelu(jnp.dot(h, e2w[...], preferred_element_type=f32) + e2b[...])

    cat = jnp.concatenate([h_struct, h_esm], axis=1)
    g = _gelu(jnp.dot(cat, g1w[...], preferred_element_type=f32) + g1b[...])
    gate = jax.nn.sigmoid(jnp.dot(g, g2w[...], preferred_element_type=f32) + g2b[...])
    h_esm = h_esm * gate

    cat2 = jnp.concatenate([h_struct, h_esm], axis=1)
    fu = _ln(cat2, fu_g[...], fu_b[...])
    y = _gelu(jnp.dot(fu, fuw[...], preferred_element_type=f32) + fub[...])
    out_ref[0] = y[:, :H]
    out_ref[1] = y[:, H:]


def _node_encoder(x_base, x_b62, x_esm, p):
    BN = 1000
    n_blocks = N_NODES // BN

    def row(d):
        return d.reshape(1, -1)

    weights = [
        row(p["base_ln"]["g"]), row(p["base_ln"]["b"]),
        p["base_l1"]["W"].T, row(p["base_l1"]["b"]),
        p["base_l2"]["W"].T, row(p["base_l2"]["b"]),
        row(p["b62_ln"]["g"]), row(p["b62_ln"]["b"]),
        p["b62_l1"]["W"].T, row(p["b62_l1"]["b"]),
        p["b62_l2"]["W"].T, row(p["b62_l2"]["b"]),
        row(p["esm_ln"]["g"]), row(p["esm_ln"]["b"]),
        p["esm_l1"]["W"].T, row(p["esm_l1"]["b"]),
        p["esm_l2"]["W"].T, row(p["esm_l2"]["b"]),
        row(p["struct_ln"]["g"]), row(p["struct_ln"]["b"]),
        p["struct_l"]["W"].T, row(p["struct_l"]["b"]),
        p["gate_l1"]["W"].T, row(p["gate_l1"]["b"]),
        p["gate_l2"]["W"].T, row(p["gate_l2"]["b"]),
        row(p["fuse_ln"]["g"]), row(p["fuse_ln"]["b"]),
        p["fuse_l"]["W"].T, row(p["fuse_l"]["b"]),
    ]

    in_specs = [
        pl.BlockSpec((BN, 12), lambda i: (i, 0)),
        pl.BlockSpec((BN, 20), lambda i: (i, 0)),
        pl.BlockSpec((BN, 1280), lambda i: (i, 0)),
    ] + [pl.BlockSpec(w.shape, lambda i: (0, 0)) for w in weights]

    return pl.pallas_call(
        _node_encoder_body,
        grid=(n_blocks,),
        in_specs=in_specs,
        out_specs=pl.BlockSpec((2, BN, H), lambda i: (0, i, 0)),
        out_shape=jax.ShapeDtypeStruct((2, N_NODES, H), jnp.float32),
    )(x_base, x_b62, x_esm, *weights)


def _make_edge_lins_body(n_lins):
    def body(dist_ref, seqbin_ref, isseq_ref, invd_ref,
             semb_ref, w1_ref, b1_ref, w2_ref, b2_ref, *rest):
        lin_ws = rest[:-1]
        out_ref = rest[-1]
        f32 = jnp.float32
        bf16 = jnp.bfloat16
        B = dist_ref.shape[0]
        dist = dist_ref[...]
        seqbin = seqbin_ref[...]
        isseq = isseq_ref[...]
        invd = invd_ref[...]

        centers = lax.broadcasted_iota(jnp.int32, (B, 32), 1).astype(f32) * (20.0 / 31.0)
        widths = 20.0 / 32.0
        gamma = 1.0 / (widths * widths + 1e-08)
        rbf = jnp.exp(-gamma * (dist - centers) ** 2)

        onehot = (seqbin == lax.broadcasted_iota(jnp.int32, (B, 9), 1)).astype(f32)
        seq_feat = jnp.dot(onehot, semb_ref[...], preferred_element_type=f32)

        raw = jnp.concatenate([rbf, seq_feat, isseq, invd], axis=1)
        hmid = _gelu(jnp.dot(raw, w1_ref[...], preferred_element_type=f32) + b1_ref[...])
        attr = jnp.dot(hmid, w2_ref[...], preferred_element_type=f32) + b2_ref[...]
        attr16 = attr.astype(bf16)

        for l in range(n_lins):
            lw, lb = lin_ws[2 * l], lin_ws[2 * l + 1]
            el = jnp.dot(attr16, lw[...], preferred_element_type=f32) + lb[...]
            out_ref[l, 0] = el[:, :H]
            out_ref[l, 1] = el[:, H:]
    return body


def _edge_lins(edge_dist, edge_seqbin, edge_is_seq, edge_inv_dist, p, layers):
    BE = 2048
    n_blocks = EPAD // BE
    col = lambda a: a.reshape(-1, 1)
    weights = [
        p["seq_emb"],
        p["edge_l1"]["W"].T, p["edge_l1"]["b"].reshape(1, -1),
        p["edge_l2"]["W"].T, p["edge_l2"]["b"].reshape(1, -1),
    ]
    for l in layers:
        c = p["convs"][l]
        weights += [c["lin"]["W"].T.astype(jnp.bfloat16), c["lin"]["b"].reshape(1, -1)]
    in_specs = [pl.BlockSpec((BE, 1), lambda i: (i, 0))] * 4 + [
        pl.BlockSpec(w.shape, lambda i: (0, 0)) for w in weights
    ]
    nl = len(layers)
    return pl.pallas_call(
        _make_edge_lins_body(nl),
        grid=(n_blocks,),
        in_specs=in_specs,
        out_specs=pl.BlockSpec((nl, 2, BE, H), lambda i: (0, 0, i, 0)),
        out_shape=jax.ShapeDtypeStruct((nl, 2, EPAD, H), jnp.float32),
    )(col(edge_dist), col(edge_seqbin.astype(jnp.int32)),
      col(edge_is_seq), col(edge_inv_dist), *weights)


def _make_mp_kernel():
    mesh = plsc.VectorSubcoreMesh(core_axis_name="c", subcore_axis_name="s")

    @functools.partial(
        pl.kernel,
        out_type=jax.ShapeDtypeStruct((2 * NPAD, H), jnp.float32),
        mesh=mesh,
        scratch_types=[
            pltpu.VMEM((IC * KE,), jnp.int32),
            pltpu.VMEM((IC, KE), jnp.int32),
            pltpu.VMEM((KE, H), jnp.float32),
            pltpu.VMEM((KE, H), jnp.float32),
            pltpu.VMEM((KE, H), jnp.float32),
            pltpu.VMEM((KE, H), jnp.float32),
            pltpu.VMEM_SHARED((NPAD, H), jnp.float32),
            pltpu.SemaphoreType.DMA,
            pltpu.SemaphoreType.DMA,
            pltpu.SemaphoreType.DMA,
            pltpu.SemaphoreType.DMA,
            pltpu.SemaphoreType.DMA,
            pltpu.SemaphoreType.DMA,
        ],
    )
    def mp(xsc, el, src2, dst3, out,
           src_i, dst_i, g0, g1, e0, e1, acc, sg0, sg1, se0, se1, ss0, ss1):
        c = lax.axis_index("c")
        s = lax.axis_index("s")
        off = c * N_NODES
        MASKHI = jnp.int32(-65536)

        zero = jnp.zeros((16,), jnp.float32)

        def z_body(r, carry):
            for t in range(H // 16):
                g0[r, pl.ds(t * 16, 16)] = zero
            return carry
        lax.fori_loop(0, KE, z_body, 0)
        for i in range(ZC):
            pltpu.sync_copy(g0, acc.at[pl.ds(s * NZ + i * KE, KE)])
        plsc.subcore_barrier()

        el_base = c * EPAD + s * EPT

        def load_idx(k):
            pltpu.sync_copy(src2.at[s].at[pl.ds(k * IC * KE, IC * KE)], src_i)

            def adj_body(r, carry):
                sl = pl.ds(r * 16, 16)
                src_i[sl] = src_i[sl] + off
                return carry
            lax.fori_loop(0, IC * KE // 16, adj_body, 0)
            pltpu.sync_copy(dst3.at[s].at[pl.ds(k * IC, IC)], dst_i)

        def start(jl, g, e, sg, se, k):
            pltpu.async_copy(xsc.at[src_i.at[pl.ds(jl * KE, KE)]], g, sg)
            pltpu.async_copy(
                el.at[pl.ds(el_base + (k * IC + jl) * KE, KE)], e, se)

        def finish(jl, g, e, sg, se, ss, k):
            pltpu.make_async_copy(xsc.at[src_i.at[pl.ds(jl * KE, KE)]], g, sg).wait()
            pltpu.make_async_copy(
                el.at[pl.ds(el_base + (k * IC + jl) * KE, KE)], e, se).wait()

            def cmp_body(q, carry):
                r = 2 * q
                for rr in (r, r + 1):
                    for t in range(H // 16):
                        sl = (rr, pl.ds(t * 16, 16))
                        g[sl] = jnp.maximum(g[sl] + e[sl], 0.0)
                return carry
            lax.fori_loop(0, KE // 2, cmp_body, 0)
            pltpu.sync_copy(g, acc.at[dst_i.at[jl]], add=True)

        for k in range(NCH):
            load_idx(k)
            start(0, g0, e0, sg0, se0, k)
            start(1, g1, e1, sg1, se1, k)

            def pair_body(t, carry):
                jl = 2 * t
                jl1 = 2 * t + 1
                finish(jl, g0, e0, sg0, se0, ss0, k)

                @pl.when(jl + 2 < IC)
                def _():
                    start(jl + 2, g0, e0, sg0, se0, k)

                finish(jl1, g1, e1, sg1, se1, ss1, k)

                @pl.when(jl1 + 2 < IC)
                def _():
                    start(jl1 + 2, g1, e1, sg1, se1, k)
                return carry
            lax.fori_loop(0, IC // 2, pair_body, 0)
        plsc.subcore_barrier()

        for i in range(ZC):
            pltpu.sync_copy(acc.at[pl.ds(s * NZ + i * KE, KE)],
                            out.at[pl.ds(c * NPAD + s * NZ + i * KE, KE)])

    return mp


_MP = _make_mp_kernel()


import numpy as _np
PERM_I = _np.empty(D, dtype=_np.int32)
for _g in range(D // 32):
    for _j in range(16):
        PERM_I[_g * 32 + 2 * _j] = _g * 32 + _j
        PERM_I[_g * 32 + 2 * _j + 1] = _g * 32 + 16 + _j


def _node_mlp_body(x_ref, agg_ref, w1, b1, w2, b2, g_ref, be_ref,
                   out_ref):
    f32 = jnp.float32
    x = jnp.concatenate([x_ref[0], x_ref[1]], axis=1)
    agg = jnp.concatenate([agg_ref[0], agg_ref[1]], axis=1)
    h = _gelu(jnp.dot(x + agg, w1[...], preferred_element_type=f32) + b1[...])
    y = jnp.dot(h, w2[...], preferred_element_type=f32) + b2[...]
    y = _ln(y, g_ref[...], be_ref[...])
    y = _gelu(y)
    y = y + x
    out_ref[0] = y[:, :H]
    out_ref[1] = y[:, H:]


def _node_mlp(x2, aggp, c, nrm):
    BN = 2000
    n_blocks = N_NODES // BN
    weights = [
        c["nn1"]["W"].T, c["nn1"]["b"].reshape(1, -1),
        c["nn2"]["W"].T, c["nn2"]["b"].reshape(1, -1),
        nrm["g"].reshape(1, -1), nrm["b"].reshape(1, -1),
    ]
    in_specs = [
        pl.BlockSpec((2, BN, H), lambda i: (0, i, 0)),
        pl.BlockSpec((2, BN, H), lambda i: (0, i, 0)),
    ] + [pl.BlockSpec(w.shape, lambda i: (0, 0)) for w in weights]
    return pl.pallas_call(
        _node_mlp_body,
        grid=(n_blocks,),
        in_specs=in_specs,
        out_specs=pl.BlockSpec((2, BN, H), lambda i: (0, i, 0)),
        out_shape=jax.ShapeDtypeStruct((2, N_NODES, H), jnp.float32),
    )(x2, aggp, *weights)


def kernel(x_base, x_b62, x_esm, edge_dist, edge_seqbin, edge_is_seq,
           edge_inv_dist, edge_index, batch, params):
    p = params

    def padv(a, value=0.0):
        return jnp.pad(a.reshape(16, N_EDGES // 16), ((0, 0), (0, EPT - N_EDGES // 16)),
                       constant_values=value).reshape(-1)

    x2 = _node_encoder(x_base, x_b62, x_esm, p)
    ed, es, ei, ev = (padv(edge_dist), padv(edge_seqbin.astype(jnp.int32)),
                      padv(edge_is_seq), padv(edge_inv_dist))
    el_all = _edge_lins(ed, es, ei, ev, p, [0, 1, 2, 3, 4])

    src2 = padv(edge_index[0].astype(jnp.int32), 0).reshape(16, EPT)
    dst3 = padv(edge_index[1].astype(jnp.int32), NPAD - 1).reshape(16, NB, KE)

    el_flat = el_all.reshape(5, 2 * EPAD, H)

    for l, (c, nrm) in enumerate(zip(p["convs"], p["norms"])):
        xsc = x2.reshape(2 * N_NODES, H)
        aggp = _MP(xsc, el_flat[l], src2, dst3)
        x2 = _node_mlp(x2, aggp.reshape(2, NPAD, H), c, nrm)

    x = jnp.concatenate([x2[0], x2[1]], axis=1)
    ones = jnp.ones((N_NODES, 1), jnp.float32)
    cnt = jax.ops.segment_sum(ones, batch, num_segments=NUM_GRAPHS)
    h_mean = jax.ops.segment_sum(x, batch, num_segments=NUM_GRAPHS) / jnp.clip(cnt, 1.0, None)
    h_max = jax.ops.segment_max(x, batch, num_segments=NUM_GRAPHS)
    pooled = jnp.concatenate([h_mean, h_max], axis=1)
    return _gelu(pooled @ p["readout"]["W"].T + p["readout"]["b"])
